# Initial kernel scaffold; baseline (speedup 1.0000x reference)
import jax
import jax.numpy as jnp
from jax.experimental import pallas as pl


def kernel(token_ids, token_table, pos_table):
    raise NotImplementedError("write your pallas kernel here")


# speedup vs baseline: 1.0126x; 1.0126x over previous
import functools

import jax
import jax.numpy as jnp
from jax import lax
from jax.experimental import pallas as pl
from jax.experimental.pallas import tpu as pltpu
from jax.experimental.pallas import tpu_sc as plsc

BATCH = 4
SEQ = 2048
D = 768
TOKENS = BATCH * SEQ
NUM_WORKERS = 32
PER_WORKER = TOKENS // NUM_WORKERS
CHUNK = 64
NUM_CHUNKS = PER_WORKER // CHUNK

_mesh = plsc.VectorSubcoreMesh(core_axis_name="c", subcore_axis_name="s")


@functools.partial(
    pl.kernel,
    out_type=jax.ShapeDtypeStruct((TOKENS, D), jnp.float32),
    mesh=_mesh,
    scratch_types=[
        pltpu.VMEM((CHUNK,), jnp.int32),
        pltpu.VMEM((CHUNK, D), jnp.float32),
        pltpu.VMEM((CHUNK, D), jnp.float32),
        pltpu.SemaphoreType.DMA,
    ],
)
def _embed(ids_hbm, table_hbm, pos_hbm, out_hbm, idx_v, buf, pbuf, sem):
    wid = lax.axis_index("s") * 2 + lax.axis_index("c")
    for c in range(NUM_CHUNKS):
        base = wid * PER_WORKER + c * CHUNK
        s0 = lax.rem(base, SEQ)
        pltpu.sync_copy(ids_hbm.at[pl.ds(base, CHUNK)], idx_v)
        gather = pltpu.async_copy(table_hbm.at[idx_v], buf, sem)
        pltpu.sync_copy(pos_hbm.at[pl.ds(s0, CHUNK)], pbuf)
        gather.wait()

        def add_row(r, _):
            for k in range(D // 16):
                sl = pl.ds(k * 16, 16)
                plsc.addupdate(buf.at[r, sl], pbuf[r, sl])
            return 0

        lax.fori_loop(0, CHUNK, add_row, 0, unroll=False)
        pltpu.sync_copy(buf, out_hbm.at[pl.ds(base, CHUNK)])


def kernel(token_ids, token_table, pos_table):
    ids_flat = token_ids.reshape(TOKENS)
    out = _embed(ids_flat, token_table, pos_table)
    return out.reshape(BATCH, SEQ, D)


# speedup vs baseline: 1.1381x; 1.1240x over previous
import jax
import jax.numpy as jnp
from jax import lax
from jax.experimental import pallas as pl
from jax.experimental.pallas import tpu as pltpu
from jax.experimental.pallas import tpu_sc as plsc

BATCH = 4
SEQ = 2048
D = 768
TOKENS = BATCH * SEQ
NUM_WORKERS = 32
S_PER_W = SEQ // NUM_WORKERS
CHUNK = 32
CH_PER_B = S_PER_W // CHUNK
NUM_CHUNKS = BATCH * CH_PER_B
NBUF = 3

_mesh = plsc.VectorSubcoreMesh(core_axis_name="c", subcore_axis_name="s")

_scratch = (
    [pltpu.VMEM((S_PER_W * BATCH,), jnp.int32)]
    + [pltpu.VMEM((S_PER_W, D), jnp.float32)]
    + [pltpu.VMEM((CHUNK, D), jnp.float32) for _ in range(NBUF)]
    + [pltpu.SemaphoreType.DMA for _ in range(NBUF)]
    + [pltpu.SemaphoreType.DMA for _ in range(NBUF)]
)


@pl.kernel(
    out_type=jax.ShapeDtypeStruct((TOKENS, D), jnp.float32),
    mesh=_mesh,
    scratch_types=_scratch,
)
def _embed(ids_hbm, table_hbm, pos_hbm, out_hbm, idx_v, pbuf,
           b0, b1, b2, g0, g1, g2, w0, w1, w2):
    bufs = (b0, b1, b2)
    gsem = (g0, g1, g2)
    wsem = (w0, w1, w2)
    wid = lax.axis_index("s") * 2 + lax.axis_index("c")
    s_base = wid * S_PER_W

    pltpu.sync_copy(pos_hbm.at[pl.ds(s_base, S_PER_W)], pbuf)
    for b in range(BATCH):
        pltpu.sync_copy(
            ids_hbm.at[pl.ds(b * SEQ + s_base, S_PER_W)],
            idx_v.at[pl.ds(b * S_PER_W, S_PER_W)],
        )

    def start_gather(j):
        slot = j % NBUF
        return pltpu.async_copy(
            table_hbm.at[idx_v.at[pl.ds(j * CHUNK, CHUNK)]],
            bufs[slot], gsem[slot],
        )

    def out_base(j):
        b, c = divmod(j, CH_PER_B)
        return b * SEQ + s_base + c * CHUNK

    gathers = [None] * NUM_CHUNKS
    writes = [None] * NUM_CHUNKS
    for j in range(min(NBUF - 1, NUM_CHUNKS)):
        gathers[j] = start_gather(j)

    for j in range(NUM_CHUNKS):
        slot = j % NBUF
        gathers[j].wait()
        buf = bufs[slot]
        prow = (j % CH_PER_B) * CHUNK

        def add_row(r, _):
            for k in range(D // 16):
                sl = pl.ds(k * 16, 16)
                plsc.addupdate(buf.at[r, sl], pbuf[prow + r, sl])
            return 0

        lax.fori_loop(0, CHUNK, add_row, 0, unroll=False)
        writes[j] = pltpu.async_copy(
            buf, out_hbm.at[pl.ds(out_base(j), CHUNK)], wsem[slot])
        nxt = j + NBUF - 1
        if nxt < NUM_CHUNKS:
            prev = nxt - NBUF
            if prev >= 0:
                writes[prev].wait()
            gathers[nxt] = start_gather(nxt)

    for j in range(max(0, NUM_CHUNKS - NBUF), NUM_CHUNKS):
        if writes[j] is not None:
            writes[j].wait()


def kernel(token_ids, token_table, pos_table):
    ids_flat = token_ids.reshape(TOKENS)
    out = _embed(ids_flat, token_table, pos_table)
    return out.reshape(BATCH, SEQ, D)


# speedup vs baseline: 1.1394x; 1.0011x over previous
import jax
import jax.numpy as jnp
from jax import lax
from jax.experimental import pallas as pl
from jax.experimental.pallas import tpu as pltpu
from jax.experimental.pallas import tpu_sc as plsc

BATCH = 4
SEQ = 2048
D = 768
TOKENS = BATCH * SEQ
NUM_WORKERS = 32
S_PER_W = SEQ // NUM_WORKERS
CHUNK = 16
CH_PER_B = S_PER_W // CHUNK
NUM_CHUNKS = BATCH * CH_PER_B
NBUF = 6
AHEAD = 4

_mesh = plsc.VectorSubcoreMesh(core_axis_name="c", subcore_axis_name="s")

_scratch = (
    [pltpu.VMEM((S_PER_W * BATCH,), jnp.int32)]
    + [pltpu.VMEM((S_PER_W, D), jnp.float32)]
    + [pltpu.VMEM((CHUNK, D), jnp.float32) for _ in range(NBUF)]
    + [pltpu.SemaphoreType.DMA for _ in range(NBUF)]
    + [pltpu.SemaphoreType.DMA for _ in range(NBUF)]
    + [pltpu.SemaphoreType.DMA]
    + [pltpu.SemaphoreType.DMA]
)


@pl.kernel(
    out_type=jax.ShapeDtypeStruct((TOKENS, D), jnp.float32),
    mesh=_mesh,
    scratch_types=_scratch,
)
def _embed(ids_hbm, table_hbm, pos_hbm, out_hbm, idx_v, pbuf,
           b0, b1, b2, b3, b4, b5,
           g0, g1, g2, g3, g4, g5,
           w0, w1, w2, w3, w4, w5, isem, psem):
    bufs = (b0, b1, b2, b3, b4, b5)
    gsem = (g0, g1, g2, g3, g4, g5)
    wsem = (w0, w1, w2, w3, w4, w5)
    wid = lax.axis_index("s") * 2 + lax.axis_index("c")
    s_base = wid * S_PER_W

    id_copies = [
        pltpu.async_copy(
            ids_hbm.at[pl.ds(b * SEQ + s_base, S_PER_W)],
            idx_v.at[pl.ds(b * S_PER_W, S_PER_W)], isem)
        for b in range(BATCH)
    ]
    pos_copy = pltpu.async_copy(pos_hbm.at[pl.ds(s_base, S_PER_W)], pbuf, psem)
    for c in id_copies:
        c.wait()

    def start_gather(j):
        slot = j % NBUF
        return pltpu.async_copy(
            table_hbm.at[idx_v.at[pl.ds(j * CHUNK, CHUNK)]],
            bufs[slot], gsem[slot],
        )

    def out_base(j):
        b, c = divmod(j, CH_PER_B)
        return b * SEQ + s_base + c * CHUNK

    gathers = [None] * NUM_CHUNKS
    writes = [None] * NUM_CHUNKS
    for j in range(min(AHEAD, NUM_CHUNKS)):
        gathers[j] = start_gather(j)
    pos_copy.wait()

    for j in range(NUM_CHUNKS):
        slot = j % NBUF
        gathers[j].wait()
        buf = bufs[slot]
        prow = (j % CH_PER_B) * CHUNK

        def add_row(r, _):
            for k in range(D // 16):
                sl = pl.ds(k * 16, 16)
                plsc.addupdate(buf.at[r, sl], pbuf[prow + r, sl])
            return 0

        lax.fori_loop(0, CHUNK, add_row, 0, unroll=False)
        writes[j] = pltpu.async_copy(
            buf, out_hbm.at[pl.ds(out_base(j), CHUNK)], wsem[slot])
        nxt = j + AHEAD
        if nxt < NUM_CHUNKS:
            prev = nxt - NBUF
            if prev >= 0:
                writes[prev].wait()
            gathers[nxt] = start_gather(nxt)

    for j in range(max(0, NUM_CHUNKS - NBUF), NUM_CHUNKS):
        writes[j].wait()


def kernel(token_ids, token_table, pos_table):
    out = _embed(token_ids.reshape(TOKENS), token_table, pos_table)
    return out.reshape(BATCH, SEQ, D)


# speedup vs baseline: 1.1984x; 1.0518x over previous
import jax
import jax.numpy as jnp
from jax import lax
from jax.experimental import pallas as pl
from jax.experimental.pallas import tpu as pltpu
from jax.experimental.pallas import tpu_sc as plsc

BATCH = 4
SEQ = 2048
D = 768
TOKENS = BATCH * SEQ
NUM_WORKERS = 32
S_PER_W = SEQ // NUM_WORKERS
CHUNK = 16
CH_PER_B = S_PER_W // CHUNK
NUM_CHUNKS = BATCH * CH_PER_B
GBUF = 3
OBUF = 3
AHEAD = 2

_mesh = plsc.VectorSubcoreMesh(core_axis_name="c", subcore_axis_name="s")

_scratch = (
    [pltpu.VMEM((S_PER_W * BATCH,), jnp.int32)]
    + [pltpu.VMEM((S_PER_W, D), jnp.float32)]
    + [pltpu.VMEM((CHUNK, D), jnp.float32) for _ in range(GBUF)]
    + [pltpu.VMEM((CHUNK, D), jnp.float32) for _ in range(OBUF)]
    + [pltpu.SemaphoreType.DMA for _ in range(GBUF)]
    + [pltpu.SemaphoreType.DMA for _ in range(OBUF)]
    + [pltpu.SemaphoreType.DMA]
    + [pltpu.SemaphoreType.DMA]
)


@pl.kernel(
    out_type=jax.ShapeDtypeStruct((TOKENS, D), jnp.float32),
    mesh=_mesh,
    scratch_types=_scratch,
)
def _embed(ids_hbm, table_hbm, pos_hbm, out_hbm, idx_v, pbuf,
           g0, g1, g2, o0, o1, o2,
           gs0, gs1, gs2, ws0, ws1, ws2, isem, psem):
    gbufs = (g0, g1, g2)
    obufs = (o0, o1, o2)
    gsem = (gs0, gs1, gs2)
    wsem = (ws0, ws1, ws2)
    wid = lax.axis_index("s") * 2 + lax.axis_index("c")
    s_base = wid * S_PER_W

    id_copies = [
        pltpu.async_copy(
            ids_hbm.at[pl.ds(b * SEQ + s_base, S_PER_W)],
            idx_v.at[pl.ds(b * S_PER_W, S_PER_W)], isem)
        for b in range(BATCH)
    ]
    pos_copy = pltpu.async_copy(pos_hbm.at[pl.ds(s_base, S_PER_W)], pbuf, psem)
    for c in id_copies:
        c.wait()

    def start_gather(j):
        slot = j % GBUF
        return pltpu.async_copy(
            table_hbm.at[idx_v.at[pl.ds(j * CHUNK, CHUNK)]],
            gbufs[slot], gsem[slot],
        )

    def out_base(j):
        b, c = divmod(j, CH_PER_B)
        return b * SEQ + s_base + c * CHUNK

    gathers = [None] * NUM_CHUNKS
    writes = [None] * NUM_CHUNKS
    for j in range(min(AHEAD, NUM_CHUNKS)):
        gathers[j] = start_gather(j)
    pos_copy.wait()

    for j in range(NUM_CHUNKS):
        gathers[j].wait()
        if j - OBUF >= 0:
            writes[j - OBUF].wait()
        gbuf = gbufs[j % GBUF]
        obuf = obufs[j % OBUF]
        prow = (j % CH_PER_B) * CHUNK

        def add_row(r, _):
            for k in range(D // 16):
                sl = pl.ds(k * 16, 16)
                obuf[r, sl] = gbuf[r, sl] + pbuf[prow + r, sl]
            return 0

        lax.fori_loop(0, CHUNK, add_row, 0, unroll=False)
        writes[j] = pltpu.async_copy(
            obuf, out_hbm.at[pl.ds(out_base(j), CHUNK)], wsem[j % OBUF])
        nxt = j + AHEAD
        if nxt < NUM_CHUNKS:
            gathers[nxt] = start_gather(nxt)

    for j in range(max(0, NUM_CHUNKS - OBUF), NUM_CHUNKS):
        writes[j].wait()


def kernel(token_ids, token_table, pos_table):
    out = _embed(token_ids.reshape(TOKENS), token_table, pos_table)
    return out.reshape(BATCH, SEQ, D)
